# baseline (speedup 1.0000x reference)
import functools

import jax
import jax.numpy as jnp
from jax import lax
from jax.experimental import pallas as pl
from jax.experimental.pallas import tpu as pltpu
from jax.experimental.pallas import tpu_sc as plsc

N = 50000
E = 800000
G = 512
EPS = 1e-5
BN_ROWS = 2000
NB = N // BN_ROWS

NSC = 16
RNG = 12800
SROWS = RNG + 8
SSTR = RNG // NSC
ZR = 25
NP = 4 * RNG
EB = 64
KSB = 16
NSB = 50
EPAD = NSC * NSB * KSB * EB
NSB0 = 25

_HI = lax.Precision.HIGHEST

_mesh = plsc.VectorSubcoreMesh(core_axis_name="c", subcore_axis_name="s",
                               num_cores=2, num_subcores=NSC)


def _deg_body(dstr, out, degS, zb1, ones_v, didx2):
    cid = lax.axis_index("c")
    sid = lax.axis_index("s")
    zstr = NP // NSC

    def zrow(i, carry):
        zb1[pl.ds(i * 16, 16)] = jnp.zeros((16,), jnp.float32)
        return carry
    lax.fori_loop(0, zstr // 16, zrow, 0)
    for k in range(EB // 16):
        ones_v[pl.ds(k * 16, 16)] = jnp.ones((16,), jnp.float32)
    pltpu.sync_copy(zb1, degS.at[pl.ds(sid * zstr, zstr)])
    plsc.subcore_barrier()

    lo = jnp.where(cid == 0, 0, NSB0)
    hi = jnp.where(cid == 0, NSB0, NSB)

    def sb(ks, carry):
        pltpu.sync_copy(dstr.at[sid, pl.ds(ks * KSB, KSB), :], didx2)
        for j in range(KSB):
            pltpu.sync_copy(ones_v, degS.at[didx2.at[j]], add=True)
        return carry
    lax.fori_loop(lo, hi, sb, 0)
    plsc.subcore_barrier()
    pltpu.sync_copy(degS.at[pl.ds(sid * zstr, zstr)],
                    out.at[cid, 0, pl.ds(sid * zstr, zstr)])


_deg_call = pl.kernel(
    _deg_body,
    out_type=jax.ShapeDtypeStruct((2, 1, NP), jnp.float32),
    mesh=_mesh,
    scratch_types=[
        pltpu.VMEM_SHARED((NP,), jnp.float32),
        pltpu.VMEM((NP // NSC,), jnp.float32),
        pltpu.VMEM((EB,), jnp.float32),
        pltpu.VMEM((KSB, EB), jnp.int32),
    ],
)


def _make_prop(ntab):
    def body(u, srcr, dstr, out, S, zb, sidx2, didx2, didxr, r0, r1, s0, s1):
        cid = lax.axis_index("c")
        sid = lax.axis_index("s")
        bufs = (r0, r1)
        sems = (s0, s1)

        def zrow(i, carry):
            for k in range(8):
                zb[i, pl.ds(k * 16, 16)] = jnp.zeros((16,), jnp.float32)
            return carry
        lax.fori_loop(0, ZR, zrow, 0)

        for chunk in range(ntab):
            for p in range(2):
                base = (2 * p + cid) * RNG
                for z in range(32):
                    pltpu.sync_copy(zb, S.at[pl.ds(sid * SSTR + z * ZR, ZR), :])
                plsc.subcore_barrier()

                def sb(ks, carry):
                    pltpu.sync_copy(srcr.at[sid, pl.ds(ks * KSB, KSB), :],
                                    sidx2)
                    pltpu.sync_copy(dstr.at[sid, pl.ds(ks * KSB, KSB), :],
                                    didx2)
                    for j in range(KSB):
                        for k in range(EB // 16):
                            d = didx2[j, pl.ds(k * 16, 16)]
                            r = d - base
                            ok = jnp.logical_and(r >= 0, r < RNG)
                            didxr[j, pl.ds(k * 16, 16)] = jnp.where(ok, r, RNG)
                    desc = pltpu.async_copy(u.at[chunk].at[sidx2.at[0]],
                                            r0, s0)
                    for j in range(KSB):
                        if j + 1 < KSB:
                            nxt = pltpu.async_copy(
                                u.at[chunk].at[sidx2.at[j + 1]],
                                bufs[(j + 1) % 2], sems[(j + 1) % 2])
                        desc.wait()
                        pltpu.sync_copy(bufs[j % 2], S.at[didxr.at[j]],
                                        add=True)
                        if j + 1 < KSB:
                            desc = nxt
                    return carry
                lax.fori_loop(0, NSB, sb, 0)
                plsc.subcore_barrier()
                off = pl.multiple_of(base + sid * SSTR, 8)
                pltpu.sync_copy(S.at[pl.ds(sid * SSTR, SSTR), :],
                                out.at[chunk, pl.ds(off, SSTR), :])

    return pl.kernel(
        body,
        out_type=jax.ShapeDtypeStruct((ntab, NP, 128), jnp.float32),
        mesh=_mesh,
        scratch_types=[
            pltpu.VMEM_SHARED((SROWS, 128), jnp.float32),
            pltpu.VMEM((ZR, 128), jnp.float32),
            pltpu.VMEM((KSB, EB), jnp.int32),
            pltpu.VMEM((KSB, EB), jnp.int32),
            pltpu.VMEM((KSB, EB), jnp.int32),
            pltpu.VMEM((EB, 128), jnp.float32),
            pltpu.VMEM((EB, 128), jnp.float32),
            pltpu.SemaphoreType.DMA,
            pltpu.SemaphoreType.DMA,
        ],
    )


_prop1t = _make_prop(1)
_prop2t = _make_prop(2)


def _prep_body(x_ref, dinv_ref, u_ref, v_ref):
    dv = dinv_ref[...]
    xp = x_ref[...]
    u_ref[0] = dv * xp
    v_ref[0] = (dv * dv) * xp


def _prep(xp, dcol):
    return pl.pallas_call(
        _prep_body,
        grid=(NB,),
        in_specs=[
            pl.BlockSpec((BN_ROWS, 128), lambda i: (i, 0)),
            pl.BlockSpec((BN_ROWS, 1), lambda i: (i, 0)),
        ],
        out_specs=[
            pl.BlockSpec((1, BN_ROWS, 128), lambda i: (0, i, 0)),
            pl.BlockSpec((1, BN_ROWS, 128), lambda i: (0, i, 0)),
        ],
        out_shape=[
            jax.ShapeDtypeStruct((1, NP, 128), jnp.float32),
            jax.ShapeDtypeStruct((1, NP, 128), jnp.float32),
        ],
    )(xp, dcol)


def _make_mm_stats(ntab, Do):
    def body(s_ref, v_ref, dinv_ref, w_ref, b_ref, y_ref, so_ref, sso_ref,
             sacc, ssacc):
        i = pl.program_id(0)
        dv = dinv_ref[...]
        acc = None
        for c in range(ntab):
            z = dv * s_ref[c] + v_ref[c]
            d = jnp.dot(z, w_ref[c], preferred_element_type=jnp.float32,
                        precision=_HI)
            acc = d if acc is None else acc + d
        y = acc + b_ref[...]
        y_ref[...] = y
        s = jnp.sum(y, axis=0, keepdims=True)
        ss = jnp.sum(y * y, axis=0, keepdims=True)

        @pl.when(i == 0)
        def _():
            sacc[...] = s
            ssacc[...] = ss

        @pl.when(i > 0)
        def _():
            sacc[...] += s
            ssacc[...] += ss

        @pl.when(i == NB - 1)
        def _():
            so_ref[...] = sacc[...]
            sso_ref[...] = ssacc[...]

    def call(scat, v, dcol, Wr, b):
        return pl.pallas_call(
            body,
            grid=(NB,),
            in_specs=[
                pl.BlockSpec((ntab, BN_ROWS, 128), lambda i: (0, i, 0)),
                pl.BlockSpec((ntab, BN_ROWS, 128), lambda i: (0, i, 0)),
                pl.BlockSpec((BN_ROWS, 1), lambda i: (i, 0)),
                pl.BlockSpec((ntab, 128, Do), lambda i: (0, 0, 0)),
                pl.BlockSpec((1, Do), lambda i: (0, 0)),
            ],
            out_specs=[
                pl.BlockSpec((BN_ROWS, Do), lambda i: (i, 0)),
                pl.BlockSpec((1, Do), lambda i: (0, 0)),
                pl.BlockSpec((1, Do), lambda i: (0, 0)),
            ],
            out_shape=[
                jax.ShapeDtypeStruct((N, Do), jnp.float32),
                jax.ShapeDtypeStruct((1, Do), jnp.float32),
                jax.ShapeDtypeStruct((1, Do), jnp.float32),
            ],
            scratch_shapes=[
                pltpu.VMEM((1, Do), jnp.float32),
                pltpu.VMEM((1, Do), jnp.float32),
            ],
        )(scat, v, dcol, Wr, b)
    return call


_mm1 = _make_mm_stats(1, 128)
_mm2 = _make_mm_stats(1, 256)
_mm3 = _make_mm_stats(2, 256)


def _make_norm_chunk(ntab, Do):
    def body(y_ref, sc_ref, sh_ref, dinv_ref, u_ref, v_ref):
        h = jnp.maximum(y_ref[...] * sc_ref[...] + sh_ref[...], 0.0)
        dv = dinv_ref[...]
        u = dv * h
        v = dv * u
        for c in range(ntab):
            u_ref[c] = u[:, c * 128:(c + 1) * 128]
            v_ref[c] = v[:, c * 128:(c + 1) * 128]

    def call(y, scale, shift, dcol):
        return pl.pallas_call(
            body,
            grid=(NB,),
            in_specs=[
                pl.BlockSpec((BN_ROWS, Do), lambda i: (i, 0)),
                pl.BlockSpec((1, Do), lambda i: (0, 0)),
                pl.BlockSpec((1, Do), lambda i: (0, 0)),
                pl.BlockSpec((BN_ROWS, 1), lambda i: (i, 0)),
            ],
            out_specs=[
                pl.BlockSpec((ntab, BN_ROWS, 128), lambda i: (0, i, 0)),
                pl.BlockSpec((ntab, BN_ROWS, 128), lambda i: (0, i, 0)),
            ],
            out_shape=[
                jax.ShapeDtypeStruct((ntab, NP, 128), jnp.float32),
                jax.ShapeDtypeStruct((ntab, NP, 128), jnp.float32),
            ],
        )(y, scale, shift, dcol)
    return call


_nc1 = _make_norm_chunk(1, 128)
_nc2 = _make_norm_chunk(2, 256)


def _pool_body(y_ref, sc_ref, sh_ref, ids_ref, wp_ref, bp_ref, out_ref,
               acc_ref, cnt_ref):
    i = pl.program_id(0)
    h = jnp.maximum(y_ref[...] * sc_ref[...] + sh_ref[...], 0.0)
    ids = ids_ref[0]
    gid = lax.broadcasted_iota(jnp.int32, (G, BN_ROWS), 0)
    oT = (gid == ids).astype(jnp.float32)
    part = jnp.dot(oT, h, preferred_element_type=jnp.float32, precision=_HI)
    ones = jnp.ones((BN_ROWS, 8), jnp.float32)
    pcnt = jnp.dot(oT, ones, preferred_element_type=jnp.float32, precision=_HI)

    @pl.when(i == 0)
    def _():
        acc_ref[...] = part
        cnt_ref[...] = pcnt

    @pl.when(i > 0)
    def _():
        acc_ref[...] += part
        cnt_ref[...] += pcnt

    @pl.when(i == NB - 1)
    def _():
        pooled = acc_ref[...] / jnp.maximum(cnt_ref[...][:, 0:1], 1.0)
        out_ref[...] = jnp.dot(pooled, wp_ref[...],
                               preferred_element_type=jnp.float32,
                               precision=_HI) + bp_ref[...]


def _pool_linear(y3, scale, shift, batch3, Wp, bp):
    Do = y3.shape[1]
    return pl.pallas_call(
        _pool_body,
        grid=(NB,),
        in_specs=[
            pl.BlockSpec((BN_ROWS, Do), lambda i: (i, 0)),
            pl.BlockSpec((1, Do), lambda i: (0, 0)),
            pl.BlockSpec((1, Do), lambda i: (0, 0)),
            pl.BlockSpec((1, 1, BN_ROWS), lambda i: (i, 0, 0)),
            pl.BlockSpec((Do, 256), lambda i: (0, 0)),
            pl.BlockSpec((1, 256), lambda i: (0, 0)),
        ],
        out_specs=pl.BlockSpec((G, 256), lambda i: (0, 0)),
        out_shape=jax.ShapeDtypeStruct((G, 256), jnp.float32),
        scratch_shapes=[
            pltpu.VMEM((G, Do), jnp.float32),
            pltpu.VMEM((G, 8), jnp.float32),
        ],
    )(y3, scale, shift, batch3, Wp, bp)


def _scale_shift(s, ss, g, be):
    m = s / N
    v = ss / N - m * m
    scale = (g[None, :] / jnp.sqrt(v + EPS))
    shift = be[None, :] - m * scale
    return scale, shift


def kernel(x, edge_index, batch, W1, b1, g1, be1, W2, b2, g2, be2,
           W3, b3, g3, be3, Wp, bp):
    pad = jnp.full((EPAD - E,), N, jnp.int32)
    srcr = jnp.concatenate([edge_index[0], pad]).reshape(NSC, NSB * KSB, EB)
    dstr = jnp.concatenate([edge_index[1], pad]).reshape(NSC, NSB * KSB, EB)

    degp = _deg_call(dstr)
    deg = degp[0, 0, :N] + degp[1, 0, :N] + 1.0
    dcol = (deg ** -0.5)[:, None]

    xp = jnp.pad(x, ((0, 0), (0, 119)))
    W1p = jnp.pad(W1, ((0, 119), (0, 0)))

    u0, v0 = _prep(xp, dcol)
    scat1 = _prop1t(u0, srcr, dstr)
    y1, s1, ss1 = _mm1(scat1, v0, dcol, W1p[None], b1[None, :])
    sc1, sh1 = _scale_shift(s1, ss1, g1, be1)
    u1, v1 = _nc1(y1, sc1, sh1, dcol)

    scat2 = _prop1t(u1, srcr, dstr)
    y2, s2, ss2 = _mm2(scat2, v1, dcol, W2[None], b2[None, :])
    sc2, sh2 = _scale_shift(s2, ss2, g2, be2)
    u2, v2 = _nc2(y2, sc2, sh2, dcol)

    scat3 = _prop2t(u2, srcr, dstr)
    y3, s3, ss3 = _mm3(scat3, v2, dcol, W3.reshape(2, 128, 256), b3[None, :])
    sc3, sh3 = _scale_shift(s3, ss3, g3, be3)

    batch3 = batch.reshape(NB, 1, BN_ROWS)
    return _pool_linear(y3, sc3, sh3, batch3, Wp, bp[None, :])
